# baseline (speedup 1.0000x reference)
import functools

import jax
import jax.numpy as jnp
from jax import lax
from jax.experimental import pallas as pl
from jax.experimental.pallas import tpu as pltpu
from jax.experimental.pallas import tpu_sc as plsc

E = 8
K = 2
H = 1024
F = 4096
B = 1
S = 2048
T = B * S
NP = T * K
TILE = 128
NT = NP // TILE + E
NSLOT = NT * TILE
NC = 2
NS = 16
NW = NC * NS
PPW = NP // NW
NCH = PPW // 32


def _router_body(x_ref, gw_ref, pe_ref, pw_ref):
    logits = lax.dot_general(x_ref[...], gw_ref[...],
                             (((1,), (1,)), ((), ())),
                             preferred_element_type=jnp.float32)
    iota = lax.broadcasted_iota(jnp.int32, (T, E), 1)
    m1 = jnp.max(logits, axis=1, keepdims=True)
    i1 = jnp.min(jnp.where(logits == m1, iota, E), axis=1, keepdims=True)
    masked = jnp.where(iota == i1, -jnp.inf, logits)
    m2 = jnp.max(masked, axis=1, keepdims=True)
    i2 = jnp.min(jnp.where(masked == m2, iota, E), axis=1, keepdims=True)
    w1 = 1.0 / (1.0 + jnp.exp(m2 - m1))
    k_iota = lax.broadcasted_iota(jnp.int32, (T, K), 1)
    pe_ref[...] = jnp.where(k_iota == 0, i1, i2)
    pw_ref[...] = jnp.where(k_iota == 0, w1, 1.0 - w1)


def _router(flat, gate_w):
    return pl.pallas_call(
        _router_body,
        out_shape=(jax.ShapeDtypeStruct((T, K), jnp.int32),
                   jax.ShapeDtypeStruct((T, K), jnp.float32)),
    )(flat, gate_w)


def _dispatch_body(pe_hbm, flat_hbm, xg_hbm, slot_hbm, te_hbm,
                   pe_v, tok_v, sl_v, rows_v, te_v, sem):
    cid = lax.axis_index("c")
    sid = lax.axis_index("s")
    wid = sid * NC + cid
    iota = lax.iota(jnp.int32, 16)
    zero = jnp.zeros((16,), jnp.int32)

    pltpu.sync_copy(pe_hbm, pe_v)

    def hist_step(j, accs):
        v = pe_v[pl.ds(j * 16, 16)]
        return tuple(accs[e] + jnp.where(v == e, 1, 0) for e in range(E))

    my0 = wid * (PPW // 16)
    accs = lax.fori_loop(0, my0, hist_step, tuple(zero for _ in range(E)))
    pref_s = [jnp.sum(a) for a in accs]
    accs = lax.fori_loop(my0, NP // 16, hist_step, accs)
    tot_s = [jnp.sum(a) for a in accs]

    def lanes(vals):
        acc = zero
        for e in range(E):
            acc = acc + jnp.where(iota == e, vals[e], 0)
        return acc

    hist = lanes(tot_s)
    padded = jnp.bitwise_and(hist + (TILE - 1), -TILE)
    offs = plsc.cumsum(padded) - padded
    run = offs + lanes(pref_s)

    base_pair = wid * PPW
    for j in range(PPW // 16):
        v = pe_v[pl.ds(base_pair + j * 16, 16)]
        slot = zero
        for e in range(E):
            m = v == e
            inc = jnp.where(m, 1, 0)
            r = plsc.cumsum(inc)
            base_e = jnp.sum(jnp.where(iota == e, run, 0))
            slot = jnp.where(m, base_e + r - 1, slot)
            run = run + jnp.where(iota == e, jnp.sum(inc), 0)
        tok = (base_pair + j * 16 + iota) >> 1
        tok_v[j // 2, pl.ds((j % 2) * 16, 16)] = tok
        sl_v[j // 2, pl.ds((j % 2) * 16, 16)] = slot

    pltpu.sync_copy(sl_v, slot_hbm.at[pl.ds(wid * NCH, NCH)])

    @pl.when(wid == 0)
    def _():
        ends = offs + padded
        maxu = jnp.max(jnp.where(hist > 0, iota, 0))
        for tv in range(3):
            ti = (iota + tv * 16) * TILE
            c = zero
            for e in range(E):
                end_e = jnp.sum(jnp.where(iota == e, ends, 0))
                c = c + jnp.where(ti >= end_e, 1, 0)
            te_v[pl.ds(tv * 16, 16)] = jnp.minimum(c, maxu)
        pltpu.sync_copy(te_v, te_hbm)

    for c in range(NCH):
        pltpu.async_copy(flat_hbm.at[tok_v.at[c]], rows_v, sem).wait()
        pltpu.async_copy(rows_v, xg_hbm.at[sl_v.at[c]], sem).wait()


def _dispatch(pe, flat):
    mesh = plsc.VectorSubcoreMesh(core_axis_name="c", subcore_axis_name="s",
                                  num_cores=NC, num_subcores=NS)
    return pl.kernel(
        _dispatch_body,
        out_type=(jax.ShapeDtypeStruct((NSLOT, H), jnp.float32),
                  jax.ShapeDtypeStruct((NP // 32, 32), jnp.int32),
                  jax.ShapeDtypeStruct((48,), jnp.int32)),
        mesh=mesh,
        scratch_types=[
            pltpu.VMEM((NP,), jnp.int32),
            pltpu.VMEM((NCH, 32), jnp.int32),
            pltpu.VMEM((NCH, 32), jnp.int32),
            pltpu.VMEM((32, H), jnp.float32),
            pltpu.VMEM((48,), jnp.int32),
            pltpu.SemaphoreType.DMA,
        ],
        compiler_params=pltpu.CompilerParams(needs_layout_passes=False),
    )(pe, flat)


NF = 2
FC = F // NF


def _ffn_body(te_ref, x_ref, w1_ref, b1_ref, w2_ref, b2_ref, o_ref,
              w1c, w2c, eid_ref):
    f = pl.program_id(0)
    i = pl.program_id(1)
    cur = te_ref[i] * NF + f

    @pl.when(jnp.logical_or(i == 0, eid_ref[0] != cur))
    def _():
        w1c[...] = w1_ref[0].astype(jnp.bfloat16)
        w2c[...] = w2_ref[0].astype(jnp.bfloat16)
        eid_ref[0] = cur

    x = x_ref[...].astype(jnp.bfloat16)
    h = jnp.dot(x, w1c[...], preferred_element_type=jnp.float32)
    h = jax.nn.gelu(h + b1_ref[0]).astype(jnp.bfloat16)
    y = jnp.dot(h, w2c[...], preferred_element_type=jnp.float32)
    o_ref[0] = y + jnp.where(f == 0, 1.0, 0.0) * b2_ref[0]


def _ffn(te, xg, W1, b1, W2, b2):
    grid_spec = pltpu.PrefetchScalarGridSpec(
        num_scalar_prefetch=1,
        grid=(NF, NT),
        in_specs=[
            pl.BlockSpec((TILE, H), lambda f, i, te: (i, 0)),
            pl.BlockSpec((1, H, FC), lambda f, i, te: (te[i], 0, f)),
            pl.BlockSpec((1, 1, FC), lambda f, i, te: (te[i], 0, f)),
            pl.BlockSpec((1, FC, H), lambda f, i, te: (te[i], f, 0)),
            pl.BlockSpec((1, 1, H), lambda f, i, te: (te[i], 0, 0)),
        ],
        out_specs=pl.BlockSpec((1, TILE, H), lambda f, i, te: (f, i, 0)),
        scratch_shapes=[
            pltpu.VMEM((H, FC), jnp.bfloat16),
            pltpu.VMEM((FC, H), jnp.bfloat16),
            pltpu.SMEM((1,), jnp.int32),
        ],
    )
    return pl.pallas_call(
        _ffn_body,
        grid_spec=grid_spec,
        out_shape=jax.ShapeDtypeStruct((NF, NSLOT, H), jnp.float32),
        compiler_params=pltpu.CompilerParams(
            vmem_limit_bytes=100 * 1024 * 1024),
    )(te, xg, W1, b1.reshape(E, 1, F), W2, b2.reshape(E, 1, H))


def _combine_body(yg_hbm, slot_hbm, pw_hbm, out_hbm,
                  sl_v, sl2_v, w_v, r0_v, r1_v, o_v, sem):
    cid = lax.axis_index("c")
    sid = lax.axis_index("s")
    wid = sid * NC + cid
    iota = lax.iota(jnp.int32, 16)

    pltpu.sync_copy(slot_hbm.at[pl.ds(wid * NCH, NCH)], sl_v)
    pltpu.sync_copy(pw_hbm.at[pl.ds(wid * NCH, NCH)], w_v)
    for c in range(NCH):
        for h in range(2):
            sl2_v[c, pl.ds(h * 16, 16)] = (
                sl_v[c, pl.ds(h * 16, 16)] + NSLOT)

    for c in range(NCH):
        cp0 = pltpu.async_copy(yg_hbm.at[sl_v.at[c]], r0_v, sem)
        cp1 = pltpu.async_copy(yg_hbm.at[sl2_v.at[c]], r1_v, sem)
        w_lo = w_v[c, pl.ds(0, 16)]
        w_hi = w_v[c, pl.ds(16, 16)]
        ws = []
        for jj in range(16):
            l = 2 * jj
            src = w_lo if l < 16 else w_hi
            w0 = jnp.sum(jnp.where(iota == (l % 16), src, 0.0))
            w1 = jnp.sum(jnp.where(iota == ((l + 1) % 16), src, 0.0))
            ws.append((w0, w1))
        cp0.wait()
        cp1.wait()

        def col_step(k, _):
            for jj in range(16):
                w0, w1 = ws[jj]
                d = pl.ds(k * 16, 16)
                o_v[jj, d] = (
                    w0 * (r0_v[2 * jj, d] + r1_v[2 * jj, d])
                    + w1 * (r0_v[2 * jj + 1, d] + r1_v[2 * jj + 1, d]))
            return 0

        lax.fori_loop(0, H // 16, col_step, 0)
        pltpu.sync_copy(o_v, out_hbm.at[pl.ds(wid * 64 + c * 16, 16)])


def _combine(yg2, slot2d, pw2d):
    mesh = plsc.VectorSubcoreMesh(core_axis_name="c", subcore_axis_name="s",
                                  num_cores=NC, num_subcores=NS)
    return pl.kernel(
        _combine_body,
        out_type=jax.ShapeDtypeStruct((T, H), jnp.float32),
        mesh=mesh,
        scratch_types=[
            pltpu.VMEM((NCH, 32), jnp.int32),
            pltpu.VMEM((NCH, 32), jnp.int32),
            pltpu.VMEM((NCH, 32), jnp.float32),
            pltpu.VMEM((32, H), jnp.float32),
            pltpu.VMEM((32, H), jnp.float32),
            pltpu.VMEM((16, H), jnp.float32),
            pltpu.SemaphoreType.DMA,
        ],
        compiler_params=pltpu.CompilerParams(needs_layout_passes=False),
    )(yg2, slot2d, pw2d)


def kernel(hidden_states, gate_w, W1, b1, W2, b2):
    flat = hidden_states.reshape(T, H)
    pe2, pw2 = _router(flat, gate_w)
    xg, slot2d, te = _dispatch(pe2.reshape(NP), flat)
    yg = _ffn(te, xg, W1, b1, W2, b2)
    out = _combine(yg.reshape(NF * NSLOT, H), slot2d,
                   pw2.reshape(NP // 32, 32))
    return out.reshape(B, S, H)


# speedup vs baseline: 1.0281x; 1.0281x over previous
import functools

import jax
import jax.numpy as jnp
from jax import lax
from jax.experimental import pallas as pl
from jax.experimental.pallas import tpu as pltpu
from jax.experimental.pallas import tpu_sc as plsc

E = 8
K = 2
H = 1024
F = 4096
B = 1
S = 2048
T = B * S
NP = T * K
TILE = 128
NT = NP // TILE + E
NSLOT = NT * TILE
NC = 2
NS = 16
NW = NC * NS
PPW = NP // NW
NCH = PPW // 32


def _router_body(x_ref, gw_ref, pe_ref, pw_ref):
    logits = lax.dot_general(x_ref[...], gw_ref[...],
                             (((1,), (1,)), ((), ())),
                             preferred_element_type=jnp.float32)
    iota = lax.broadcasted_iota(jnp.int32, (T, E), 1)
    m1 = jnp.max(logits, axis=1, keepdims=True)
    i1 = jnp.min(jnp.where(logits == m1, iota, E), axis=1, keepdims=True)
    masked = jnp.where(iota == i1, -jnp.inf, logits)
    m2 = jnp.max(masked, axis=1, keepdims=True)
    i2 = jnp.min(jnp.where(masked == m2, iota, E), axis=1, keepdims=True)
    w1 = 1.0 / (1.0 + jnp.exp(m2 - m1))
    k_iota = lax.broadcasted_iota(jnp.int32, (T, K), 1)
    pe_ref[...] = jnp.where(k_iota == 0, i1, i2)
    pw_ref[...] = jnp.where(k_iota == 0, w1, 1.0 - w1)


def _router(flat, gate_w):
    return pl.pallas_call(
        _router_body,
        out_shape=(jax.ShapeDtypeStruct((T, K), jnp.int32),
                   jax.ShapeDtypeStruct((T, K), jnp.float32)),
    )(flat, gate_w)


def _dispatch_body(pe_hbm, flat_hbm, xg_hbm, slot_hbm, te_hbm,
                   pe_v, tok_v, sl_v, rows_v, te_v, sem_g, sem_s0, sem_s1):
    cid = lax.axis_index("c")
    sid = lax.axis_index("s")
    wid = sid * NC + cid
    iota = lax.iota(jnp.int32, 16)
    zero = jnp.zeros((16,), jnp.int32)

    pltpu.sync_copy(pe_hbm, pe_v)

    def hist_step(j, accs):
        v = pe_v[pl.ds(j * 16, 16)]
        return tuple(accs[e] + jnp.where(v == e, 1, 0) for e in range(E))

    my0 = wid * (PPW // 16)
    accs = lax.fori_loop(0, my0, hist_step, tuple(zero for _ in range(E)))
    pref_s = [jnp.sum(a) for a in accs]
    accs = lax.fori_loop(my0, NP // 16, hist_step, accs)
    tot_s = [jnp.sum(a) for a in accs]

    def lanes(vals):
        acc = zero
        for e in range(E):
            acc = acc + jnp.where(iota == e, vals[e], 0)
        return acc

    hist = lanes(tot_s)
    padded = jnp.bitwise_and(hist + (TILE - 1), -TILE)
    offs = plsc.cumsum(padded) - padded
    run = offs + lanes(pref_s)

    base_pair = wid * PPW
    for j in range(PPW // 16):
        v = pe_v[pl.ds(base_pair + j * 16, 16)]
        slot = zero
        for e in range(E):
            m = v == e
            inc = jnp.where(m, 1, 0)
            r = plsc.cumsum(inc)
            base_e = jnp.sum(jnp.where(iota == e, run, 0))
            slot = jnp.where(m, base_e + r - 1, slot)
            run = run + jnp.where(iota == e, jnp.sum(inc), 0)
        tok = (base_pair + j * 16 + iota) >> 1
        tok_v[j // 2, pl.ds((j % 2) * 16, 16)] = tok
        sl_v[j // 2, pl.ds((j % 2) * 16, 16)] = slot

    pltpu.sync_copy(sl_v, slot_hbm.at[pl.ds(wid * NCH, NCH)])

    @pl.when(wid == 0)
    def _():
        ends = offs + padded
        maxu = jnp.max(jnp.where(hist > 0, iota, 0))
        for tv in range(3):
            ti = (iota + tv * 16) * TILE
            c = zero
            for e in range(E):
                end_e = jnp.sum(jnp.where(iota == e, ends, 0))
                c = c + jnp.where(ti >= end_e, 1, 0)
            te_v[pl.ds(tv * 16, 16)] = jnp.minimum(c, maxu)
        pltpu.sync_copy(te_v, te_hbm)

    ssem = (sem_s0, sem_s1)
    scat = [None, None]
    g = pltpu.async_copy(flat_hbm.at[tok_v.at[0]], rows_v.at[0], sem_g)
    for c in range(NCH):
        g.wait()
        scat[c % 2] = pltpu.async_copy(rows_v.at[c % 2],
                                       xg_hbm.at[sl_v.at[c]], ssem[c % 2])
        if c + 1 < NCH:
            if c >= 1:
                scat[(c + 1) % 2].wait()
            g = pltpu.async_copy(flat_hbm.at[tok_v.at[c + 1]],
                                 rows_v.at[(c + 1) % 2], sem_g)
    scat[(NCH - 1) % 2].wait()
    scat[NCH % 2].wait()


def _dispatch(pe, flat):
    mesh = plsc.VectorSubcoreMesh(core_axis_name="c", subcore_axis_name="s",
                                  num_cores=NC, num_subcores=NS)
    return pl.kernel(
        _dispatch_body,
        out_type=(jax.ShapeDtypeStruct((NSLOT, H), jnp.float32),
                  jax.ShapeDtypeStruct((NP // 32, 32), jnp.int32),
                  jax.ShapeDtypeStruct((48,), jnp.int32)),
        mesh=mesh,
        scratch_types=[
            pltpu.VMEM((NP,), jnp.int32),
            pltpu.VMEM((NCH, 32), jnp.int32),
            pltpu.VMEM((NCH, 32), jnp.int32),
            pltpu.VMEM((2, 32, H), jnp.float32),
            pltpu.VMEM((48,), jnp.int32),
            pltpu.SemaphoreType.DMA,
            pltpu.SemaphoreType.DMA,
            pltpu.SemaphoreType.DMA,
        ],
        compiler_params=pltpu.CompilerParams(needs_layout_passes=False),
    )(pe, flat)


NF = 2
FC = F // NF


def _ffn_body(te_ref, x_ref, w1_ref, b1_ref, w2_ref, b2_ref, o_ref):
    f = pl.program_id(0)
    x = x_ref[...].astype(jnp.bfloat16)
    h = jnp.dot(x, w1_ref[0].astype(jnp.bfloat16),
                preferred_element_type=jnp.float32)
    h = jax.nn.gelu(h + b1_ref[0]).astype(jnp.bfloat16)
    y = jnp.dot(h, w2_ref[0].astype(jnp.bfloat16),
                preferred_element_type=jnp.float32)
    o_ref[0] = y + jnp.where(f == 0, 1.0, 0.0) * b2_ref[0]


def _ffn(te, xg, W1, b1, W2, b2):
    grid_spec = pltpu.PrefetchScalarGridSpec(
        num_scalar_prefetch=1,
        grid=(NF, NT),
        in_specs=[
            pl.BlockSpec((TILE, H), lambda f, i, te: (i, 0)),
            pl.BlockSpec((1, H, FC), lambda f, i, te: (te[i], 0, f)),
            pl.BlockSpec((1, 1, FC), lambda f, i, te: (te[i], 0, f)),
            pl.BlockSpec((1, FC, H), lambda f, i, te: (te[i], f, 0)),
            pl.BlockSpec((1, 1, H), lambda f, i, te: (te[i], 0, 0)),
        ],
        out_specs=pl.BlockSpec((1, TILE, H), lambda f, i, te: (f, i, 0)),
    )
    return pl.pallas_call(
        _ffn_body,
        grid_spec=grid_spec,
        out_shape=jax.ShapeDtypeStruct((NF, NSLOT, H), jnp.float32),
        compiler_params=pltpu.CompilerParams(
            vmem_limit_bytes=100 * 1024 * 1024),
    )(te, xg, W1, b1.reshape(E, 1, F), W2, b2.reshape(E, 1, H))


def _combine_body(yg_hbm, slot_hbm, pw_hbm, out_hbm,
                  sl_v, sl2_v, w_v, r0_v, r1_v, o_v, sem):
    cid = lax.axis_index("c")
    sid = lax.axis_index("s")
    wid = sid * NC + cid
    iota = lax.iota(jnp.int32, 16)

    pltpu.sync_copy(slot_hbm.at[pl.ds(wid * NCH, NCH)], sl_v)
    pltpu.sync_copy(pw_hbm.at[pl.ds(wid * NCH, NCH)], w_v)
    for c in range(NCH):
        for h in range(2):
            sl2_v[c, pl.ds(h * 16, 16)] = (
                sl_v[c, pl.ds(h * 16, 16)] + NSLOT)

    for c in range(NCH):
        cp0 = pltpu.async_copy(yg_hbm.at[sl_v.at[c]], r0_v, sem)
        cp1 = pltpu.async_copy(yg_hbm.at[sl2_v.at[c]], r1_v, sem)
        w_lo = w_v[c, pl.ds(0, 16)]
        w_hi = w_v[c, pl.ds(16, 16)]
        ws = []
        for jj in range(16):
            l = 2 * jj
            src = w_lo if l < 16 else w_hi
            w0 = jnp.sum(jnp.where(iota == (l % 16), src, 0.0))
            w1 = jnp.sum(jnp.where(iota == ((l + 1) % 16), src, 0.0))
            ws.append((w0, w1))
        cp0.wait()
        cp1.wait()

        def col_step(k, _):
            for jj in range(16):
                w0, w1 = ws[jj]
                d = pl.ds(k * 16, 16)
                o_v[jj, d] = (
                    w0 * (r0_v[2 * jj, d] + r1_v[2 * jj, d])
                    + w1 * (r0_v[2 * jj + 1, d] + r1_v[2 * jj + 1, d]))
            return 0

        lax.fori_loop(0, H // 16, col_step, 0)
        pltpu.sync_copy(o_v, out_hbm.at[pl.ds(wid * 64 + c * 16, 16)])


def _combine(yg2, slot2d, pw2d):
    mesh = plsc.VectorSubcoreMesh(core_axis_name="c", subcore_axis_name="s",
                                  num_cores=NC, num_subcores=NS)
    return pl.kernel(
        _combine_body,
        out_type=jax.ShapeDtypeStruct((T, H), jnp.float32),
        mesh=mesh,
        scratch_types=[
            pltpu.VMEM((NCH, 32), jnp.int32),
            pltpu.VMEM((NCH, 32), jnp.int32),
            pltpu.VMEM((NCH, 32), jnp.float32),
            pltpu.VMEM((32, H), jnp.float32),
            pltpu.VMEM((32, H), jnp.float32),
            pltpu.VMEM((16, H), jnp.float32),
            pltpu.SemaphoreType.DMA,
        ],
        compiler_params=pltpu.CompilerParams(needs_layout_passes=False),
    )(yg2, slot2d, pw2d)


def kernel(hidden_states, gate_w, W1, b1, W2, b2):
    flat = hidden_states.reshape(T, H)
    pe2, pw2 = _router(flat, gate_w)
    xg, slot2d, te = _dispatch(pe2.reshape(NP), flat)
    yg = _ffn(te, xg, W1, b1, W2, b2)
    out = _combine(yg.reshape(NF * NSLOT, H), slot2d,
                   pw2.reshape(NP // 32, 32))
    return out.reshape(B, S, H)


# speedup vs baseline: 1.1128x; 1.0824x over previous
import functools

import jax
import jax.numpy as jnp
from jax import lax
from jax.experimental import pallas as pl
from jax.experimental.pallas import tpu as pltpu
from jax.experimental.pallas import tpu_sc as plsc

E = 8
K = 2
H = 1024
F = 4096
B = 1
S = 2048
T = B * S
NP = T * K
TILE = 128
NT = NP // TILE + E
NSLOT = NT * TILE
NC = 2
NS = 16
NW = NC * NS
PPW = NP // NW
NCH = PPW // 32


def _router_body(x_ref, gw_ref, pe_ref, pw_ref):
    logits = lax.dot_general(x_ref[...], gw_ref[...],
                             (((1,), (1,)), ((), ())),
                             preferred_element_type=jnp.float32)
    iota = lax.broadcasted_iota(jnp.int32, (T, E), 1)
    m1 = jnp.max(logits, axis=1, keepdims=True)
    i1 = jnp.min(jnp.where(logits == m1, iota, E), axis=1, keepdims=True)
    masked = jnp.where(iota == i1, -jnp.inf, logits)
    m2 = jnp.max(masked, axis=1, keepdims=True)
    i2 = jnp.min(jnp.where(masked == m2, iota, E), axis=1, keepdims=True)
    w1 = 1.0 / (1.0 + jnp.exp(m2 - m1))
    k_iota = lax.broadcasted_iota(jnp.int32, (T, K), 1)
    pe_ref[...] = jnp.where(k_iota == 0, i1, i2)
    pw_ref[...] = jnp.where(k_iota == 0, w1, 1.0 - w1)


def _router(flat, gate_w):
    return pl.pallas_call(
        _router_body,
        out_shape=(jax.ShapeDtypeStruct((T, K), jnp.int32),
                   jax.ShapeDtypeStruct((T, K), jnp.float32)),
    )(flat, gate_w)


def _dispatch_body(pe_hbm, flat_hbm, xg_hbm, slot_hbm, te_hbm,
                   pe_v, tok_v, sl_v, rows_v, te_v, sem_g, sem_s0, sem_s1):
    cid = lax.axis_index("c")
    sid = lax.axis_index("s")
    wid = sid * NC + cid
    iota = lax.iota(jnp.int32, 16)
    zero = jnp.zeros((16,), jnp.int32)

    pltpu.sync_copy(pe_hbm, pe_v)

    def hist_step(j, accs):
        v = pe_v[pl.ds(j * 16, 16)]
        return tuple(accs[e] + jnp.where(v == e, 1, 0) for e in range(E))

    my0 = wid * (PPW // 16)
    accs = lax.fori_loop(0, my0, hist_step, tuple(zero for _ in range(E)))
    pref_s = [jnp.sum(a) for a in accs]
    accs = lax.fori_loop(my0, NP // 16, hist_step, accs)
    tot_s = [jnp.sum(a) for a in accs]

    def lanes(vals):
        acc = zero
        for e in range(E):
            acc = acc + jnp.where(iota == e, vals[e], 0)
        return acc

    hist = lanes(tot_s)
    padded = jnp.bitwise_and(hist + (TILE - 1), -TILE)
    offs = plsc.cumsum(padded) - padded
    run = offs + lanes(pref_s)

    base_pair = wid * PPW
    for j in range(PPW // 16):
        v = pe_v[pl.ds(base_pair + j * 16, 16)]
        slot = zero
        for e in range(E):
            m = v == e
            inc = jnp.where(m, 1, 0)
            r = plsc.cumsum(inc)
            base_e = jnp.sum(jnp.where(iota == e, run, 0))
            slot = jnp.where(m, base_e + r - 1, slot)
            run = run + jnp.where(iota == e, jnp.sum(inc), 0)
        tok = (base_pair + j * 16 + iota) >> 1
        tok_v[j // 2, pl.ds((j % 2) * 16, 16)] = tok
        sl_v[j // 2, pl.ds((j % 2) * 16, 16)] = slot

    pltpu.sync_copy(sl_v, slot_hbm.at[pl.ds(wid * NCH, NCH)])

    @pl.when(wid == 0)
    def _():
        ends = offs + padded
        maxu = jnp.max(jnp.where(hist > 0, iota, 0))
        for tv in range(3):
            ti = (iota + tv * 16) * TILE
            c = zero
            for e in range(E):
                end_e = jnp.sum(jnp.where(iota == e, ends, 0))
                c = c + jnp.where(ti >= end_e, 1, 0)
            te_v[pl.ds(tv * 16, 16)] = jnp.minimum(c, maxu)
        pltpu.sync_copy(te_v, te_hbm)

    ssem = (sem_s0, sem_s1)
    scat = [None, None]
    g = pltpu.async_copy(flat_hbm.at[tok_v.at[0]], rows_v.at[0], sem_g)
    for c in range(NCH):
        g.wait()
        scat[c % 2] = pltpu.async_copy(rows_v.at[c % 2],
                                       xg_hbm.at[sl_v.at[c]], ssem[c % 2])
        if c + 1 < NCH:
            if c >= 1:
                scat[(c + 1) % 2].wait()
            g = pltpu.async_copy(flat_hbm.at[tok_v.at[c + 1]],
                                 rows_v.at[(c + 1) % 2], sem_g)
    scat[(NCH - 1) % 2].wait()
    scat[NCH % 2].wait()


def _dispatch(pe, flat):
    mesh = plsc.VectorSubcoreMesh(core_axis_name="c", subcore_axis_name="s",
                                  num_cores=NC, num_subcores=NS)
    return pl.kernel(
        _dispatch_body,
        out_type=(jax.ShapeDtypeStruct((NSLOT, H), jnp.float32),
                  jax.ShapeDtypeStruct((NP // 32, 32), jnp.int32),
                  jax.ShapeDtypeStruct((48,), jnp.int32)),
        mesh=mesh,
        scratch_types=[
            pltpu.VMEM((NP,), jnp.int32),
            pltpu.VMEM((NCH, 32), jnp.int32),
            pltpu.VMEM((NCH, 32), jnp.int32),
            pltpu.VMEM((2, 32, H), jnp.float32),
            pltpu.VMEM((48,), jnp.int32),
            pltpu.SemaphoreType.DMA,
            pltpu.SemaphoreType.DMA,
            pltpu.SemaphoreType.DMA,
        ],
        compiler_params=pltpu.CompilerParams(needs_layout_passes=False),
    )(pe, flat)


NF = 2
FC = F // NF
NSPL = 4


def _ffn_body(te_ref, x_ref, w1_hbm, b1_ref, w2_hbm, b2_ref, o_ref,
              w1s, w2s, slot_ref, sems):
    f = pl.program_id(0)
    i = pl.program_id(1)
    HS = H // NSPL
    FS = FC // NSPL

    def block_copies(e, ff, s):
        cps = []
        for k in range(NSPL):
            cps.append(pltpu.make_async_copy(
                w1_hbm.at[e, pl.ds(k * HS, HS), pl.ds(ff * FC, FC)],
                w1s.at[s, pl.ds(k * HS, HS)], sems.at[s, k]))
            cps.append(pltpu.make_async_copy(
                w2_hbm.at[e, pl.ds(ff * FC + k * FS, FS), :],
                w2s.at[s, pl.ds(k * FS, FS)], sems.at[s, NSPL + k]))
        return cps

    e_cur = te_ref[i]
    is_tr = jnp.logical_or(i == 0, e_cur != te_ref[jnp.maximum(i - 1, 0)])

    @pl.when(is_tr)
    def _():
        first = jnp.logical_and(f == 0, i == 0)
        s = jnp.where(first, 0, 1 - slot_ref[0])

        @pl.when(first)
        def _():
            for cp in block_copies(e_cur, f, 0):
                cp.start()

        slot_ref[0] = s
        for cp in block_copies(e_cur, f, s):
            cp.wait()

        j = lax.while_loop(
            lambda j: jnp.logical_and(j < NT, te_ref[jnp.minimum(j, NT - 1)]
                                      == e_cur),
            lambda j: j + 1, i + 1)
        has_next = jnp.logical_or(j < NT, f + 1 < NF)
        e_n = jnp.where(j < NT, te_ref[jnp.minimum(j, NT - 1)], te_ref[0])
        f_n = jnp.where(j < NT, f, f + 1)

        @pl.when(has_next)
        def _():
            for cp in block_copies(e_n, f_n, 1 - s):
                cp.start()

    s = slot_ref[0]
    x = x_ref[...].astype(jnp.bfloat16)
    h = jnp.dot(x, w1s[s].astype(jnp.bfloat16),
                preferred_element_type=jnp.float32)
    h = jax.nn.gelu(h + b1_ref[0]).astype(jnp.bfloat16)
    y = jnp.dot(h, w2s[s].astype(jnp.bfloat16),
                preferred_element_type=jnp.float32)
    o_ref[0] = y + jnp.where(f == 0, 1.0, 0.0) * b2_ref[0]


def _ffn(te, xg, W1, b1, W2, b2):
    grid_spec = pltpu.PrefetchScalarGridSpec(
        num_scalar_prefetch=1,
        grid=(NF, NT),
        in_specs=[
            pl.BlockSpec((TILE, H), lambda f, i, te: (i, 0)),
            pl.BlockSpec(memory_space=pl.ANY),
            pl.BlockSpec((1, 1, FC), lambda f, i, te: (te[i], 0, f)),
            pl.BlockSpec(memory_space=pl.ANY),
            pl.BlockSpec((1, 1, H), lambda f, i, te: (te[i], 0, 0)),
        ],
        out_specs=pl.BlockSpec((1, TILE, H), lambda f, i, te: (f, i, 0)),
        scratch_shapes=[
            pltpu.VMEM((2, H, FC), jnp.float32),
            pltpu.VMEM((2, FC, H), jnp.float32),
            pltpu.SMEM((1,), jnp.int32),
            pltpu.SemaphoreType.DMA((2, 2 * NSPL)),
        ],
    )
    return pl.pallas_call(
        _ffn_body,
        grid_spec=grid_spec,
        out_shape=jax.ShapeDtypeStruct((NF, NSLOT, H), jnp.float32),
        compiler_params=pltpu.CompilerParams(
            vmem_limit_bytes=100 * 1024 * 1024),
    )(te, xg, W1, b1.reshape(E, 1, F), W2, b2.reshape(E, 1, H))


def _combine_body(yg_hbm, slot_hbm, pw_hbm, out_hbm,
                  sl_v, sl2_v, w_v, r0_v, r1_v, o_v, sem):
    cid = lax.axis_index("c")
    sid = lax.axis_index("s")
    wid = sid * NC + cid
    iota = lax.iota(jnp.int32, 16)

    pltpu.sync_copy(slot_hbm.at[pl.ds(wid * NCH, NCH)], sl_v)
    pltpu.sync_copy(pw_hbm.at[pl.ds(wid * NCH, NCH)], w_v)
    for c in range(NCH):
        for h in range(2):
            sl2_v[c, pl.ds(h * 16, 16)] = (
                sl_v[c, pl.ds(h * 16, 16)] + NSLOT)

    for c in range(NCH):
        cp0 = pltpu.async_copy(yg_hbm.at[sl_v.at[c]], r0_v, sem)
        cp1 = pltpu.async_copy(yg_hbm.at[sl2_v.at[c]], r1_v, sem)
        w_lo = w_v[c, pl.ds(0, 16)]
        w_hi = w_v[c, pl.ds(16, 16)]
        ws = []
        for jj in range(16):
            l = 2 * jj
            src = w_lo if l < 16 else w_hi
            w0 = jnp.sum(jnp.where(iota == (l % 16), src, 0.0))
            w1 = jnp.sum(jnp.where(iota == ((l + 1) % 16), src, 0.0))
            ws.append((w0, w1))
        cp0.wait()
        cp1.wait()

        def col_step(k, _):
            for jj in range(16):
                w0, w1 = ws[jj]
                d = pl.ds(k * 16, 16)
                o_v[jj, d] = (
                    w0 * (r0_v[2 * jj, d] + r1_v[2 * jj, d])
                    + w1 * (r0_v[2 * jj + 1, d] + r1_v[2 * jj + 1, d]))
            return 0

        lax.fori_loop(0, H // 16, col_step, 0)
        pltpu.sync_copy(o_v, out_hbm.at[pl.ds(wid * 64 + c * 16, 16)])


def _combine(yg2, slot2d, pw2d):
    mesh = plsc.VectorSubcoreMesh(core_axis_name="c", subcore_axis_name="s",
                                  num_cores=NC, num_subcores=NS)
    return pl.kernel(
        _combine_body,
        out_type=jax.ShapeDtypeStruct((T, H), jnp.float32),
        mesh=mesh,
        scratch_types=[
            pltpu.VMEM((NCH, 32), jnp.int32),
            pltpu.VMEM((NCH, 32), jnp.int32),
            pltpu.VMEM((NCH, 32), jnp.float32),
            pltpu.VMEM((32, H), jnp.float32),
            pltpu.VMEM((32, H), jnp.float32),
            pltpu.VMEM((16, H), jnp.float32),
            pltpu.SemaphoreType.DMA,
        ],
        compiler_params=pltpu.CompilerParams(needs_layout_passes=False),
    )(yg2, slot2d, pw2d)


def kernel(hidden_states, gate_w, W1, b1, W2, b2):
    flat = hidden_states.reshape(T, H)
    pe2, pw2 = _router(flat, gate_w)
    xg, slot2d, te = _dispatch(pe2.reshape(NP), flat)
    yg = _ffn(te, xg, W1, b1, W2, b2)
    out = _combine(yg.reshape(NF * NSLOT, H), slot2d,
                   pw2.reshape(NP // 32, 32))
    return out.reshape(B, S, H)


# speedup vs baseline: 1.1762x; 1.0570x over previous
import functools

import jax
import jax.numpy as jnp
from jax import lax
from jax.experimental import pallas as pl
from jax.experimental.pallas import tpu as pltpu
from jax.experimental.pallas import tpu_sc as plsc

E = 8
K = 2
H = 1024
F = 4096
B = 1
S = 2048
T = B * S
NP = T * K
TILE = 128
NT = NP // TILE + E
NSLOT = NT * TILE
NC = 2
NS = 16
NW = NC * NS
PPW = NP // NW
NCH = PPW // 32


def _router_body(x_ref, gw_ref, pe_ref, pw_ref):
    logits = lax.dot_general(x_ref[...], gw_ref[...],
                             (((1,), (1,)), ((), ())),
                             preferred_element_type=jnp.float32)
    iota = lax.broadcasted_iota(jnp.int32, (T, E), 1)
    m1 = jnp.max(logits, axis=1, keepdims=True)
    i1 = jnp.min(jnp.where(logits == m1, iota, E), axis=1, keepdims=True)
    masked = jnp.where(iota == i1, -jnp.inf, logits)
    m2 = jnp.max(masked, axis=1, keepdims=True)
    i2 = jnp.min(jnp.where(masked == m2, iota, E), axis=1, keepdims=True)
    w1 = 1.0 / (1.0 + jnp.exp(m2 - m1))
    k_iota = lax.broadcasted_iota(jnp.int32, (T, K), 1)
    pe_ref[...] = jnp.where(k_iota == 0, i1, i2)
    pw_ref[...] = jnp.where(k_iota == 0, w1, 1.0 - w1)


def _router(flat, gate_w):
    return pl.pallas_call(
        _router_body,
        out_shape=(jax.ShapeDtypeStruct((T, K), jnp.int32),
                   jax.ShapeDtypeStruct((T, K), jnp.float32)),
    )(flat, gate_w)


def _dispatch_body(pe_hbm, flat_hbm, xg_hbm, slot_hbm, te_hbm,
                   pe_v, tok_v, sl_v, rows_v, te_v, sem_g, sem_s0, sem_s1):
    cid = lax.axis_index("c")
    sid = lax.axis_index("s")
    wid = sid * NC + cid
    iota = lax.iota(jnp.int32, 16)
    zero = jnp.zeros((16,), jnp.int32)

    pltpu.sync_copy(pe_hbm, pe_v)

    def hist_step(j, accs):
        v = pe_v[pl.ds(j * 16, 16)]
        return tuple(accs[e] + jnp.where(v == e, 1, 0) for e in range(E))

    my0 = wid * (PPW // 16)
    accs = lax.fori_loop(0, my0, hist_step, tuple(zero for _ in range(E)))
    pref_s = [jnp.sum(a) for a in accs]
    accs = lax.fori_loop(my0, NP // 16, hist_step, accs)
    tot_s = [jnp.sum(a) for a in accs]

    def lanes(vals):
        acc = zero
        for e in range(E):
            acc = acc + jnp.where(iota == e, vals[e], 0)
        return acc

    hist = lanes(tot_s)
    padded = jnp.bitwise_and(hist + (TILE - 1), -TILE)
    offs = plsc.cumsum(padded) - padded
    run = offs + lanes(pref_s)

    base_pair = wid * PPW
    for j in range(PPW // 16):
        v = pe_v[pl.ds(base_pair + j * 16, 16)]
        slot = zero
        for e in range(E):
            m = v == e
            inc = jnp.where(m, 1, 0)
            r = plsc.cumsum(inc)
            base_e = jnp.sum(jnp.where(iota == e, run, 0))
            slot = jnp.where(m, base_e + r - 1, slot)
            run = run + jnp.where(iota == e, jnp.sum(inc), 0)
        tok = (base_pair + j * 16 + iota) >> 1
        tok_v[j // 2, pl.ds((j % 2) * 16, 16)] = tok
        sl_v[j // 2, pl.ds((j % 2) * 16, 16)] = slot

    pltpu.sync_copy(sl_v, slot_hbm.at[pl.ds(wid * NCH, NCH)])

    @pl.when(wid == 0)
    def _():
        ends = offs + padded
        maxu = jnp.max(jnp.where(hist > 0, iota, 0))
        for tv in range(3):
            ti = (iota + tv * 16) * TILE
            c = zero
            for e in range(E):
                end_e = jnp.sum(jnp.where(iota == e, ends, 0))
                c = c + jnp.where(ti >= end_e, 1, 0)
            te_v[pl.ds(tv * 16, 16)] = jnp.minimum(c, maxu)
        pltpu.sync_copy(te_v, te_hbm)

    ssem = (sem_s0, sem_s1)
    scat = [None, None]
    g = pltpu.async_copy(flat_hbm.at[tok_v.at[0]], rows_v.at[0], sem_g)
    for c in range(NCH):
        g.wait()
        scat[c % 2] = pltpu.async_copy(rows_v.at[c % 2],
                                       xg_hbm.at[sl_v.at[c]], ssem[c % 2])
        if c + 1 < NCH:
            if c >= 1:
                scat[(c + 1) % 2].wait()
            g = pltpu.async_copy(flat_hbm.at[tok_v.at[c + 1]],
                                 rows_v.at[(c + 1) % 2], sem_g)
    scat[(NCH - 1) % 2].wait()
    scat[NCH % 2].wait()


def _dispatch(pe, flat):
    mesh = plsc.VectorSubcoreMesh(core_axis_name="c", subcore_axis_name="s",
                                  num_cores=NC, num_subcores=NS)
    return pl.kernel(
        _dispatch_body,
        out_type=(jax.ShapeDtypeStruct((NSLOT, H), jnp.float32),
                  jax.ShapeDtypeStruct((NP // 32, 32), jnp.int32),
                  jax.ShapeDtypeStruct((48,), jnp.int32)),
        mesh=mesh,
        scratch_types=[
            pltpu.VMEM((NP,), jnp.int32),
            pltpu.VMEM((NCH, 32), jnp.int32),
            pltpu.VMEM((NCH, 32), jnp.int32),
            pltpu.VMEM((2, 32, H), jnp.float32),
            pltpu.VMEM((48,), jnp.int32),
            pltpu.SemaphoreType.DMA,
            pltpu.SemaphoreType.DMA,
            pltpu.SemaphoreType.DMA,
        ],
        compiler_params=pltpu.CompilerParams(needs_layout_passes=False),
    )(pe, flat)


NF = 2
FC = F // NF
NSPL = 4


def _ffn_body(te_ref, x_ref, w1_hbm, b1_ref, w2_hbm, b2_ref, o_hbm,
              w1s, w2s, acc, slot_ref, sems, osems):
    f = pl.program_id(0)
    i = pl.program_id(1)
    HS = H // NSPL
    FS = FC // NSPL

    def block_copies(e, ff, s):
        cps = []
        for k in range(NSPL):
            cps.append(pltpu.make_async_copy(
                w1_hbm.at[e, pl.ds(k * HS, HS), pl.ds(ff * FC, FC)],
                w1s.at[s, pl.ds(k * HS, HS)], sems.at[s, k]))
            cps.append(pltpu.make_async_copy(
                w2_hbm.at[e, pl.ds(ff * FC + k * FS, FS), :],
                w2s.at[s, pl.ds(k * FS, FS)], sems.at[s, NSPL + k]))
        return cps

    def out_copy(jj):
        return pltpu.make_async_copy(
            acc.at[pl.ds(jj * TILE, TILE)],
            o_hbm.at[pl.ds(jj * TILE, TILE)], osems.at[jj % 4])

    e_cur = te_ref[i]
    is_tr = jnp.logical_or(i == 0, e_cur != te_ref[jnp.maximum(i - 1, 0)])

    @pl.when(is_tr)
    def _():
        first = jnp.logical_and(f == 0, i == 0)
        s = jnp.where(first, 0, 1 - slot_ref[0])

        @pl.when(first)
        def _():
            for cp in block_copies(e_cur, f, 0):
                cp.start()

        slot_ref[0] = s
        for cp in block_copies(e_cur, f, s):
            cp.wait()

        j = lax.while_loop(
            lambda j: jnp.logical_and(j < NT, te_ref[jnp.minimum(j, NT - 1)]
                                      == e_cur),
            lambda j: j + 1, i + 1)
        has_next = jnp.logical_or(j < NT, f + 1 < NF)
        e_n = jnp.where(j < NT, te_ref[jnp.minimum(j, NT - 1)], te_ref[0])
        f_n = jnp.where(j < NT, f, f + 1)

        @pl.when(has_next)
        def _():
            for cp in block_copies(e_n, f_n, 1 - s):
                cp.start()

    s = slot_ref[0]
    x = x_ref[...].astype(jnp.bfloat16)
    h = jnp.dot(x, w1s[s].astype(jnp.bfloat16),
                preferred_element_type=jnp.float32)
    h = jax.nn.gelu(h + b1_ref[0]).astype(jnp.bfloat16)
    y = jnp.dot(h, w2s[s].astype(jnp.bfloat16),
                preferred_element_type=jnp.float32)

    @pl.when(f == 0)
    def _():
        acc[pl.ds(i * TILE, TILE)] = y + b2_ref[0]

    @pl.when(f == 1)
    def _():
        acc[pl.ds(i * TILE, TILE)] = acc[pl.ds(i * TILE, TILE)] + y
        out_copy(i).start()

        @pl.when(i >= 4)
        def _():
            out_copy(i - 4).wait()

    @pl.when(jnp.logical_and(f == NF - 1, i == NT - 1))
    def _():
        for jj in range(NT - 4, NT):
            out_copy(jj).wait()


def _ffn(te, xg, W1, b1, W2, b2):
    grid_spec = pltpu.PrefetchScalarGridSpec(
        num_scalar_prefetch=1,
        grid=(NF, NT),
        in_specs=[
            pl.BlockSpec((TILE, H), lambda f, i, te: (i, 0)),
            pl.BlockSpec(memory_space=pl.ANY),
            pl.BlockSpec((1, 1, FC), lambda f, i, te: (te[i], 0, f)),
            pl.BlockSpec(memory_space=pl.ANY),
            pl.BlockSpec((1, 1, H), lambda f, i, te: (te[i], 0, 0)),
        ],
        out_specs=pl.BlockSpec(memory_space=pl.ANY),
        scratch_shapes=[
            pltpu.VMEM((2, H, FC), jnp.float32),
            pltpu.VMEM((2, FC, H), jnp.float32),
            pltpu.VMEM((NSLOT, H), jnp.float32),
            pltpu.SMEM((1,), jnp.int32),
            pltpu.SemaphoreType.DMA((2, 2 * NSPL)),
            pltpu.SemaphoreType.DMA((4,)),
        ],
    )
    return pl.pallas_call(
        _ffn_body,
        grid_spec=grid_spec,
        out_shape=jax.ShapeDtypeStruct((NSLOT, H), jnp.float32),
        compiler_params=pltpu.CompilerParams(
            vmem_limit_bytes=100 * 1024 * 1024),
    )(te, xg, W1, b1.reshape(E, 1, F), W2, b2.reshape(E, 1, H))


def _combine_body(yg_hbm, slot_hbm, pw_hbm, out_hbm,
                  sl_v, w_v, r0_v, o_v, sem):
    cid = lax.axis_index("c")
    sid = lax.axis_index("s")
    wid = sid * NC + cid
    iota = lax.iota(jnp.int32, 16)

    pltpu.sync_copy(slot_hbm.at[pl.ds(wid * NCH, NCH)], sl_v)
    pltpu.sync_copy(pw_hbm.at[pl.ds(wid * NCH, NCH)], w_v)

    for c in range(NCH):
        cp0 = pltpu.async_copy(yg_hbm.at[sl_v.at[c]], r0_v, sem)
        w_lo = w_v[c, pl.ds(0, 16)]
        w_hi = w_v[c, pl.ds(16, 16)]
        ws = []
        for jj in range(16):
            l = 2 * jj
            src = w_lo if l < 16 else w_hi
            w0 = jnp.sum(jnp.where(iota == (l % 16), src, 0.0))
            w1 = jnp.sum(jnp.where(iota == ((l + 1) % 16), src, 0.0))
            ws.append((w0, w1))
        cp0.wait()

        def col_step(k, _):
            for jj in range(16):
                w0, w1 = ws[jj]
                d = pl.ds(k * 16, 16)
                o_v[jj, d] = (w0 * r0_v[2 * jj, d]
                              + w1 * r0_v[2 * jj + 1, d])
            return 0

        lax.fori_loop(0, H // 16, col_step, 0)
        pltpu.sync_copy(o_v, out_hbm.at[pl.ds(wid * 64 + c * 16, 16)])


def _combine(yg, slot2d, pw2d):
    mesh = plsc.VectorSubcoreMesh(core_axis_name="c", subcore_axis_name="s",
                                  num_cores=NC, num_subcores=NS)
    return pl.kernel(
        _combine_body,
        out_type=jax.ShapeDtypeStruct((T, H), jnp.float32),
        mesh=mesh,
        scratch_types=[
            pltpu.VMEM((NCH, 32), jnp.int32),
            pltpu.VMEM((NCH, 32), jnp.float32),
            pltpu.VMEM((32, H), jnp.float32),
            pltpu.VMEM((16, H), jnp.float32),
            pltpu.SemaphoreType.DMA,
        ],
        compiler_params=pltpu.CompilerParams(needs_layout_passes=False),
    )(yg, slot2d, pw2d)


def kernel(hidden_states, gate_w, W1, b1, W2, b2):
    flat = hidden_states.reshape(T, H)
    pe2, pw2 = _router(flat, gate_w)
    xg, slot2d, te = _dispatch(pe2.reshape(NP), flat)
    yg = _ffn(te, xg, W1, b1, W2, b2)
    out = _combine(yg, slot2d, pw2.reshape(NP // 32, 32))
    return out.reshape(B, S, H)
